# baseline (speedup 1.0000x reference)
import functools

import jax
import jax.numpy as jnp
from jax import lax
from jax.experimental import pallas as pl
from jax.experimental.pallas import tpu as pltpu
from jax.experimental.pallas import tpu_sc as plsc

_B = 16384
_D = 64
_NW = 32
_BPW = _B // _NW
_CH = 128
_NCH = _BPW // _CH
_L = 16


@functools.partial(
    pl.kernel,
    out_type=jax.ShapeDtypeStruct((_NW, _BPW, _D), jnp.float32),
    mesh=plsc.VectorSubcoreMesh(core_axis_name="c", subcore_axis_name="s"),
    scratch_types=[
        pltpu.VMEM((_BPW,), jnp.int32),
        pltpu.VMEM((_NCH, _CH, _D), jnp.float32),
        pltpu.SemaphoreType.DMA,
    ],
    compiler_params=pltpu.CompilerParams(use_tc_tiling_on_sc=False),
)
def _gather_sc(lab_hbm, cent_hbm, out_hbm, idx_v, rows_v, sem):
    wid = lax.axis_index("s") * 2 + lax.axis_index("c")

    pltpu.sync_copy(lab_hbm.at[pl.ds(wid * _BPW, _BPW)], idx_v)
    handles = [
        pltpu.async_copy(cent_hbm.at[idx_v.at[pl.ds(j * _CH, _CH)]],
                         rows_v.at[j], sem)
        for j in range(_NCH)
    ]
    for j in range(_NCH):
        handles[j].wait()
        pltpu.sync_copy(rows_v.at[j], out_hbm.at[wid, pl.ds(j * _CH, _CH)])


@functools.partial(
    pl.kernel,
    out_type=jax.ShapeDtypeStruct((_NW, _L), jnp.float32),
    mesh=plsc.VectorSubcoreMesh(core_axis_name="c", subcore_axis_name="s"),
    scratch_types=[
        pltpu.VMEM((_BPW * _D,), jnp.float32),
        pltpu.VMEM((_BPW, _D), jnp.float32),
        pltpu.VMEM((_L,), jnp.float32),
    ],
)
def _loss_sc(feat_hbm, gath_hbm, out_hbm, gath_v, feat_v, acc_v):
    wid = lax.axis_index("s") * 2 + lax.axis_index("c")

    pltpu.sync_copy(gath_hbm.at[pl.ds(wid * _BPW * _D, _BPW * _D)], gath_v)
    pltpu.sync_copy(feat_hbm.at[pl.ds(wid * _BPW, _BPW)], feat_v)

    def body(k, a):
        for ci in range(_D // _L):
            f = feat_v[k, pl.ds(ci * _L, _L)]
            c = gath_v[pl.ds(k * _D + ci * _L, _L)]
            d = f - c
            a = a + d * d
        return a

    acc = lax.fori_loop(0, _BPW, body, jnp.zeros((_L,), jnp.float32))
    acc_v[...] = acc
    pltpu.sync_copy(acc_v, out_hbm.at[wid])


def kernel(features, labels, centers):
    batch = features.shape[0]
    lab = labels.astype(jnp.int32)
    gathered = _gather_sc(lab, centers)
    partials = _loss_sc(features, gathered.reshape(-1))
    return jnp.sum(partials) / 2.0 / batch
